# baseline (speedup 1.0000x reference)
import jax
import jax.numpy as jnp
from jax.experimental import pallas as pl
from jax.experimental.pallas import tpu as pltpu

NUM_PATCHES = 1025
HIDDEN_DIM = 1280
MAX_NUM_TILES = 4


def _body(idx_ref, gate_ref, x_ref, emb_ref, tt_ref, o_ref):
    t = jnp.tanh(gate_ref[0])
    o_ref[...] = x_ref[...] + (1.0 - t) * emb_ref[...] + t * tt_ref[...]


def kernel(x, aspect_ratio_ids, tile_indices, embedding, gate, tile_table):
    bt = x.shape[0]
    pb = 208
    npb = -(-NUM_PATCHES // pb)
    idx = aspect_ratio_ids.astype(jnp.int32) * MAX_NUM_TILES + tile_indices.astype(jnp.int32)
    tt = tile_table.reshape(-1, NUM_PATCHES, HIDDEN_DIM)
    grid_spec = pltpu.PrefetchScalarGridSpec(
        num_scalar_prefetch=2,
        grid=(npb, bt),
        in_specs=[
            pl.BlockSpec((1, pb, HIDDEN_DIM), lambda i, j, idx_ref, g_ref: (j, i, 0)),
            pl.BlockSpec((pb, HIDDEN_DIM), lambda i, j, idx_ref, g_ref: (i, 0)),
            pl.BlockSpec((1, pb, HIDDEN_DIM), lambda i, j, idx_ref, g_ref: (idx_ref[j], i, 0)),
        ],
        out_specs=pl.BlockSpec((1, pb, HIDDEN_DIM), lambda i, j, idx_ref, g_ref: (j, i, 0)),
    )
    return pl.pallas_call(
        _body,
        grid_spec=grid_spec,
        out_shape=jax.ShapeDtypeStruct(x.shape, x.dtype),
        compiler_params=pltpu.CompilerParams(
            dimension_semantics=("arbitrary", "arbitrary"),
        ),
    )(idx, gate, x, embedding, tt)


# speedup vs baseline: 1.0546x; 1.0546x over previous
import jax
import jax.numpy as jnp
from jax import lax
from jax.experimental import pallas as pl
from jax.experimental.pallas import tpu as pltpu
from jax.experimental.pallas import tpu_sc as plsc

NUM_PATCHES = 1025
HIDDEN_DIM = 1280
MAX_NUM_TILES = 4
NUM_TABLE_ROWS = 9
SLAB = NUM_PATCHES * HIDDEN_DIM
NCHUNK = 41
CHUNK = SLAB // NCHUNK
CROWS = CHUNK // 128
NS = 16


def _sc_body(x_hbm, emb_hbm, tt_hbm, idx_hbm, coef_hbm, out_hbm,
             xbuf, ebuf, tbuf, idx_v, cbuf, sem):
    b = lax.axis_index("c") * NS + lax.axis_index("s")
    pltpu.sync_copy(idx_hbm.at[b], idx_v)
    pltpu.sync_copy(coef_hbm, cbuf)
    c0 = cbuf[0]
    c1 = cbuf[1]

    def chunk_body(c, carry):
        gather = pltpu.async_copy(tt_hbm.at[idx_v.at[c]], tbuf, sem)
        pltpu.sync_copy(x_hbm.at[b, c], xbuf)
        pltpu.sync_copy(emb_hbm.at[c], ebuf)
        gather.wait()

        def row(i, rcarry):
            for j in range(8):
                sl = pl.ds(16 * j, 16)
                xbuf[i, sl] = xbuf[i, sl] + c0 * ebuf[i, sl] + c1 * tbuf[0, i, sl]
            return rcarry

        lax.fori_loop(0, CROWS, row, 0)
        pltpu.sync_copy(xbuf, out_hbm.at[b, c])
        return carry

    lax.fori_loop(0, NCHUNK, chunk_body, 0)


def kernel(x, aspect_ratio_ids, tile_indices, embedding, gate, tile_table):
    bt = x.shape[0]
    t = jnp.tanh(gate)[0]
    coef = jnp.stack([jnp.full((16,), 1.0, jnp.float32) - t,
                      jnp.full((16,), 0.0, jnp.float32) + t])
    base = (aspect_ratio_ids.astype(jnp.int32) * MAX_NUM_TILES
            + tile_indices.astype(jnp.int32)) * NCHUNK
    idx_mat = base[:, None, None] + jnp.arange(NCHUNK, dtype=jnp.int32)[None, :, None]

    x4 = x.reshape(bt, NCHUNK, CROWS, 128)
    emb4 = embedding.reshape(NCHUNK, CROWS, 128)
    tt4 = tile_table.reshape(NUM_TABLE_ROWS * MAX_NUM_TILES * NCHUNK, CROWS, 128)

    mesh = plsc.VectorSubcoreMesh(core_axis_name="c", subcore_axis_name="s")
    out = pl.kernel(
        _sc_body,
        out_type=jax.ShapeDtypeStruct((bt, NCHUNK, CROWS, 128), jnp.float32),
        mesh=mesh,
        scratch_types=[
            pltpu.VMEM((CROWS, 128), jnp.float32),
            pltpu.VMEM((CROWS, 128), jnp.float32),
            pltpu.VMEM((1, CROWS, 128), jnp.float32),
            pltpu.VMEM((NCHUNK, 1), jnp.int32),
            pltpu.VMEM((2, 16), jnp.float32),
            pltpu.SemaphoreType.DMA,
        ],
    )(x4, emb4, tt4, idx_mat, coef)
    return out.reshape(bt, NUM_PATCHES, HIDDEN_DIM)
